# baseline (speedup 1.0000x reference)
import functools

import jax
import jax.numpy as jnp
from jax import lax
from jax.experimental import pallas as pl
from jax.experimental.pallas import tpu as pltpu
from jax.experimental.pallas import tpu_sc as plsc

_B = 16384
_D = 64
_NC = 2
_NS = 16
_NW = _NC * _NS
_BPW = _B // _NW
_C = 32
_NCH = _BPW // _C
_L = 16


def _sc_gather_body(uidx_hbm, iidx_hbm, utab_hbm, itab_hbm,
                    uout_hbm, iout_hbm,
                    uidx_v, iidx_v, sem_u, sem_i):
    wid = lax.axis_index("s") * _NC + lax.axis_index("c")
    base = wid * _BPW
    pltpu.sync_copy(uidx_hbm.at[pl.ds(base, _BPW)], uidx_v)
    pltpu.sync_copy(iidx_hbm.at[pl.ds(base, _BPW)], iidx_v)
    lane = lax.iota(jnp.int32, _L)

    def group(k, carry):
        uv = uidx_v[pl.ds(k * _L, _L)]
        iv = iidx_v[pl.ds(k * _L, _L)]
        for l in range(_L):
            us = jnp.max(jnp.where(lane == l, uv, 0))
            is_ = jnp.max(jnp.where(lane == l, iv, 0))
            r = k * _L + l
            pltpu.async_copy(utab_hbm.at[pl.ds(us, 1)],
                             uout_hbm.at[pl.ds(base + r, 1)], sem_u)
            pltpu.async_copy(itab_hbm.at[pl.ds(is_, 1)],
                             iout_hbm.at[pl.ds(base + r, 1)], sem_i)
        return carry

    lax.fori_loop(0, _BPW // _L, group, 0)

    def drain(r, carry):
        pltpu.make_async_copy(utab_hbm.at[pl.ds(0, 1)],
                              uout_hbm.at[pl.ds(base + r, 1)], sem_u).wait()
        pltpu.make_async_copy(itab_hbm.at[pl.ds(0, 1)],
                              iout_hbm.at[pl.ds(base + r, 1)], sem_i).wait()
        return carry

    lax.fori_loop(0, _BPW, drain, 0)


@functools.cache
def _sc_gather():
    return pl.kernel(
        _sc_gather_body,
        mesh=plsc.VectorSubcoreMesh(core_axis_name="c", subcore_axis_name="s"),
        compiler_params=pltpu.CompilerParams(needs_layout_passes=False),
        out_type=[
            jax.ShapeDtypeStruct((_B, _D), jnp.float32),
            jax.ShapeDtypeStruct((_B, _D), jnp.float32),
        ],
        scratch_types=[
            pltpu.VMEM((_BPW,), jnp.int32),
            pltpu.VMEM((_BPW,), jnp.int32),
            pltpu.SemaphoreType.DMA,
            pltpu.SemaphoreType.DMA,
        ],
    )


_BLK = 2048


def _mlp_body(u_ref, i_ref, w1u_ref, w1i_ref, b1_ref, w2_ref, b2_ref,
              w3_ref, b3_ref, wo_ref, bo_ref, out_ref):
    f32 = jnp.float32
    h = jnp.dot(u_ref[...], w1u_ref[...], preferred_element_type=f32)
    h += jnp.dot(i_ref[...], w1i_ref[...], preferred_element_type=f32)
    h = jnp.maximum(h + b1_ref[...], 0.0)
    h = jnp.maximum(jnp.dot(h, w2_ref[...], preferred_element_type=f32)
                    + b2_ref[...], 0.0)
    h = jnp.maximum(jnp.dot(h, w3_ref[...], preferred_element_type=f32)
                    + b3_ref[...], 0.0)
    z = jnp.sum(h * wo_ref[...], axis=1, keepdims=True) + bo_ref[...]
    out_ref[...] = 4.0 / (1.0 + jnp.exp(-z)) + 1.0


def _tc_mlp(u_emb, i_emb, w1u, w1i, b1, w2, b2, w3, b3, wo, bo):
    nblk = _B // _BLK
    full = lambda shape: pl.BlockSpec(shape, lambda i: (0, 0))
    return pl.pallas_call(
        _mlp_body,
        grid=(nblk,),
        in_specs=[
            pl.BlockSpec((_BLK, _D), lambda i: (i, 0)),
            pl.BlockSpec((_BLK, _D), lambda i: (i, 0)),
            full((_D, 128)),
            full((_D, 128)),
            full((1, 128)),
            full((128, 64)),
            full((1, 64)),
            full((64, 32)),
            full((1, 32)),
            full((1, 32)),
            full((1, 1)),
        ],
        out_specs=pl.BlockSpec((_BLK, 1), lambda i: (i, 0)),
        out_shape=jax.ShapeDtypeStruct((_B, 1), jnp.float32),
    )(u_emb, i_emb, w1u, w1i, b1, w2, b2, w3, b3, wo, bo)


def kernel(user_indices, item_indices, user_table, item_table,
           W1, b1, W2, b2, W3, b3, Wo, bo):
    u_emb, i_emb = _sc_gather()(user_indices.astype(jnp.int32),
                                item_indices.astype(jnp.int32),
                                user_table, item_table)
    w1u = W1[:, :_D].T
    w1i = W1[:, _D:].T
    return _tc_mlp(u_emb, i_emb, w1u, w1i,
                   b1.reshape(1, 128), W2.T, b2.reshape(1, 64),
                   W3.T, b3.reshape(1, 32), Wo.reshape(1, 32),
                   bo.reshape(1, 1))


# speedup vs baseline: 1.6870x; 1.6870x over previous
import functools

import jax
import jax.numpy as jnp
from jax import lax
from jax.experimental import pallas as pl
from jax.experimental.pallas import tpu as pltpu
from jax.experimental.pallas import tpu_sc as plsc

_B = 16384
_D = 64
_NC = 2
_NS = 16
_NW = _NC * _NS
_BPW = _B // _NW
_C = 32
_NCH = _BPW // _C
_L = 16


def _sc_gather_body(uidx_hbm, iidx_hbm, utab_hbm, itab_hbm,
                    uout_hbm, iout_hbm,
                    uidx_v, iidx_v, urows_v, irows_v, sem_u, sem_i):
    wid = lax.axis_index("s") * _NC + lax.axis_index("c")
    base = wid * _BPW
    pltpu.sync_copy(uidx_hbm.at[pl.ds(base, _BPW)], uidx_v)
    pltpu.sync_copy(iidx_hbm.at[pl.ds(base, _BPW)], iidx_v)
    lane = lax.iota(jnp.int32, _L)
    half = _BPW // 2

    for ph in range(2):
        hoff = ph * half

        def group(k, carry):
            uv = uidx_v[pl.ds(hoff + k * _L, _L)]
            iv = iidx_v[pl.ds(hoff + k * _L, _L)]
            for l in range(_L):
                us = jnp.max(jnp.where(lane == l, uv, 0))
                is_ = jnp.max(jnp.where(lane == l, iv, 0))
                r = k * _L + l
                pltpu.async_copy(utab_hbm.at[pl.ds(us, 1)],
                                 urows_v.at[pl.ds(r, 1)], sem_u)
                pltpu.async_copy(itab_hbm.at[pl.ds(is_, 1)],
                                 irows_v.at[pl.ds(r, 1)], sem_i)
            return carry

        lax.fori_loop(0, half // _L, group, 0)

        def drain(r, carry):
            pltpu.make_async_copy(utab_hbm.at[pl.ds(0, 1)],
                                  urows_v.at[pl.ds(r, 1)], sem_u).wait()
            pltpu.make_async_copy(itab_hbm.at[pl.ds(0, 1)],
                                  irows_v.at[pl.ds(r, 1)], sem_i).wait()
            return carry

        lax.fori_loop(0, half, drain, 0)
        pltpu.sync_copy(urows_v, uout_hbm.at[pl.ds(base + hoff, half)])
        pltpu.sync_copy(irows_v, iout_hbm.at[pl.ds(base + hoff, half)])


@functools.cache
def _sc_gather():
    return pl.kernel(
        _sc_gather_body,
        mesh=plsc.VectorSubcoreMesh(core_axis_name="c", subcore_axis_name="s"),
        compiler_params=pltpu.CompilerParams(needs_layout_passes=False),
        out_type=[
            jax.ShapeDtypeStruct((_B, _D), jnp.float32),
            jax.ShapeDtypeStruct((_B, _D), jnp.float32),
        ],
        scratch_types=[
            pltpu.VMEM((_BPW,), jnp.int32),
            pltpu.VMEM((_BPW,), jnp.int32),
            pltpu.VMEM((_BPW // 2, _D), jnp.float32),
            pltpu.VMEM((_BPW // 2, _D), jnp.float32),
            pltpu.SemaphoreType.DMA,
            pltpu.SemaphoreType.DMA,
        ],
    )


_BLK = 2048


def _mlp_body(u_ref, i_ref, w1u_ref, w1i_ref, b1_ref, w2_ref, b2_ref,
              w3_ref, b3_ref, wo_ref, bo_ref, out_ref):
    f32 = jnp.float32
    h = jnp.dot(u_ref[...], w1u_ref[...], preferred_element_type=f32)
    h += jnp.dot(i_ref[...], w1i_ref[...], preferred_element_type=f32)
    h = jnp.maximum(h + b1_ref[...], 0.0)
    h = jnp.maximum(jnp.dot(h, w2_ref[...], preferred_element_type=f32)
                    + b2_ref[...], 0.0)
    h = jnp.maximum(jnp.dot(h, w3_ref[...], preferred_element_type=f32)
                    + b3_ref[...], 0.0)
    z = jnp.sum(h * wo_ref[...], axis=1, keepdims=True) + bo_ref[...]
    out_ref[...] = 4.0 / (1.0 + jnp.exp(-z)) + 1.0


def _tc_mlp(u_emb, i_emb, w1u, w1i, b1, w2, b2, w3, b3, wo, bo):
    nblk = _B // _BLK
    full = lambda shape: pl.BlockSpec(shape, lambda i: (0, 0))
    return pl.pallas_call(
        _mlp_body,
        grid=(nblk,),
        in_specs=[
            pl.BlockSpec((_BLK, _D), lambda i: (i, 0)),
            pl.BlockSpec((_BLK, _D), lambda i: (i, 0)),
            full((_D, 128)),
            full((_D, 128)),
            full((1, 128)),
            full((128, 64)),
            full((1, 64)),
            full((64, 32)),
            full((1, 32)),
            full((1, 32)),
            full((1, 1)),
        ],
        out_specs=pl.BlockSpec((_BLK, 1), lambda i: (i, 0)),
        out_shape=jax.ShapeDtypeStruct((_B, 1), jnp.float32),
    )(u_emb, i_emb, w1u, w1i, b1, w2, b2, w3, b3, wo, bo)


def kernel(user_indices, item_indices, user_table, item_table,
           W1, b1, W2, b2, W3, b3, Wo, bo):
    u_emb, i_emb = _sc_gather()(user_indices.astype(jnp.int32),
                                item_indices.astype(jnp.int32),
                                user_table, item_table)
    w1u = W1[:, :_D].T
    w1i = W1[:, _D:].T
    return _tc_mlp(u_emb, i_emb, w1u, w1i,
                   b1.reshape(1, 128), W2.T, b2.reshape(1, 64),
                   W3.T, b3.reshape(1, 32), Wo.reshape(1, 32),
                   bo.reshape(1, 1))


# speedup vs baseline: 2.8398x; 1.6833x over previous
import functools

import jax
import jax.numpy as jnp
from jax import lax
from jax.experimental import pallas as pl
from jax.experimental.pallas import tpu as pltpu
from jax.experimental.pallas import tpu_sc as plsc

_B = 16384
_D = 64
_NC = 2
_NS = 16
_NW = _NC * _NS
_BPW = _B // _NW
_PH = 128
_NBUF = 4


def _scalar_at(idx_v, lane, j):
    jm = lax.rem(j, 16)
    grp = j - jm
    v = idx_v[pl.ds(grp, 16)]
    return jnp.max(jnp.where(lane == jm, v, 0))


def _sc_gather_body(uidx_hbm, iidx_hbm, utab_t, itab_t,
                    uout_hbm, iout_hbm,
                    uidx_v, iidx_v, ublk0, ublk1, ublk2, ublk3,
                    iblk0, iblk1, iblk2, iblk3, urows, irows,
                    su0, su1, su2, su3, si0, si1, si2, si3):
    wid = lax.axis_index("s") * _NC + lax.axis_index("c")
    base = wid * _BPW
    pltpu.sync_copy(uidx_hbm.at[pl.ds(base, _BPW)], uidx_v)
    pltpu.sync_copy(iidx_hbm.at[pl.ds(base, _BPW)], iidx_v)
    lane = lax.iota(jnp.int32, 16)
    ublks = [ublk0, ublk1, ublk2, ublk3]
    iblks = [iblk0, iblk1, iblk2, iblk3]
    usems = [su0, su1, su2, su3]
    isems = [si0, si1, si2, si3]

    def fetch(idx_v, tab, blk, sem, hoff, j):
        s = _scalar_at(idx_v, lane, hoff + j)
        c = pl.multiple_of(s - lax.rem(s, 128), 128)
        pltpu.async_copy(tab.at[:, pl.ds(c, 128)], blk, sem)

    def extract(idx_v, blk, rows, hoff, j):
        s = _scalar_at(idx_v, lane, hoff + j)
        l = lax.rem(s, 128)
        lv = jnp.full((16,), 0, jnp.int32) + l
        jv = jnp.full((16,), 0, jnp.int32) + j
        for g in range(4):
            dv = lane + (g * 16)
            vals = plsc.load_gather(blk, [dv, lv])
            plsc.store_scatter(rows, [jv, dv], vals)

    for ph in range(_BPW // _PH):
        hoff = ph * _PH
        for b in range(_NBUF):
            fetch(uidx_v, utab_t, ublks[b], usems[b], hoff, b)
            fetch(iidx_v, itab_t, iblks[b], isems[b], hoff, b)

        def quad(q, carry):
            for b in range(_NBUF):
                j = q * _NBUF + b
                pltpu.make_async_copy(utab_t.at[:, pl.ds(0, 128)],
                                      ublks[b], usems[b]).wait()
                extract(uidx_v, ublks[b], urows, hoff, j)
                pltpu.make_async_copy(itab_t.at[:, pl.ds(0, 128)],
                                      iblks[b], isems[b]).wait()
                extract(iidx_v, iblks[b], irows, hoff, j)

                @pl.when(j + _NBUF < _PH)
                def _():
                    fetch(uidx_v, utab_t, ublks[b], usems[b], hoff, j + _NBUF)
                    fetch(iidx_v, itab_t, iblks[b], isems[b], hoff, j + _NBUF)
            return carry

        lax.fori_loop(0, _PH // _NBUF, quad, 0)
        pltpu.sync_copy(urows, uout_hbm.at[pl.ds(base + hoff, _PH)])
        pltpu.sync_copy(irows, iout_hbm.at[pl.ds(base + hoff, _PH)])


@functools.cache
def _sc_gather():
    blk = pltpu.VMEM((_D, 128), jnp.float32)
    sem = pltpu.SemaphoreType.DMA
    return pl.kernel(
        _sc_gather_body,
        mesh=plsc.VectorSubcoreMesh(core_axis_name="c", subcore_axis_name="s"),
        compiler_params=pltpu.CompilerParams(needs_layout_passes=False),
        out_type=[
            jax.ShapeDtypeStruct((_B, _D), jnp.float32),
            jax.ShapeDtypeStruct((_B, _D), jnp.float32),
        ],
        scratch_types=[
            pltpu.VMEM((_BPW,), jnp.int32),
            pltpu.VMEM((_BPW,), jnp.int32),
            blk, blk, blk, blk, blk, blk, blk, blk,
            pltpu.VMEM((_PH, _D), jnp.float32),
            pltpu.VMEM((_PH, _D), jnp.float32),
            sem, sem, sem, sem, sem, sem, sem, sem,
        ],
    )


_BLK = 2048


def _mlp_body(u_ref, i_ref, w1u_ref, w1i_ref, b1_ref, w2_ref, b2_ref,
              w3_ref, b3_ref, wo_ref, bo_ref, out_ref):
    f32 = jnp.float32
    h = jnp.dot(u_ref[...], w1u_ref[...], preferred_element_type=f32)
    h += jnp.dot(i_ref[...], w1i_ref[...], preferred_element_type=f32)
    h = jnp.maximum(h + b1_ref[...], 0.0)
    h = jnp.maximum(jnp.dot(h, w2_ref[...], preferred_element_type=f32)
                    + b2_ref[...], 0.0)
    h = jnp.maximum(jnp.dot(h, w3_ref[...], preferred_element_type=f32)
                    + b3_ref[...], 0.0)
    z = jnp.sum(h * wo_ref[...], axis=1, keepdims=True) + bo_ref[...]
    out_ref[...] = 4.0 / (1.0 + jnp.exp(-z)) + 1.0


def _tc_mlp(u_emb, i_emb, w1u, w1i, b1, w2, b2, w3, b3, wo, bo):
    nblk = _B // _BLK
    full = lambda shape: pl.BlockSpec(shape, lambda i: (0, 0))
    return pl.pallas_call(
        _mlp_body,
        grid=(nblk,),
        in_specs=[
            pl.BlockSpec((_BLK, _D), lambda i: (i, 0)),
            pl.BlockSpec((_BLK, _D), lambda i: (i, 0)),
            full((_D, 128)),
            full((_D, 128)),
            full((1, 128)),
            full((128, 64)),
            full((1, 64)),
            full((64, 32)),
            full((1, 32)),
            full((1, 32)),
            full((1, 1)),
        ],
        out_specs=pl.BlockSpec((_BLK, 1), lambda i: (i, 0)),
        out_shape=jax.ShapeDtypeStruct((_B, 1), jnp.float32),
    )(u_emb, i_emb, w1u, w1i, b1, w2, b2, w3, b3, wo, bo)


def kernel(user_indices, item_indices, user_table, item_table,
           W1, b1, W2, b2, W3, b3, Wo, bo):
    u_emb, i_emb = _sc_gather()(user_indices.astype(jnp.int32),
                                item_indices.astype(jnp.int32),
                                user_table.T, item_table.T)
    w1u = W1[:, :_D].T
    w1i = W1[:, _D:].T
    return _tc_mlp(u_emb, i_emb, w1u, w1i,
                   b1.reshape(1, 128), W2.T, b2.reshape(1, 64),
                   W3.T, b3.reshape(1, 32), Wo.reshape(1, 32),
                   bo.reshape(1, 1))


# speedup vs baseline: 2.8447x; 1.0018x over previous
import functools

import jax
import jax.numpy as jnp
from jax import lax
from jax.experimental import pallas as pl
from jax.experimental.pallas import tpu as pltpu
from jax.experimental.pallas import tpu_sc as plsc

_B = 16384
_D = 64
_NC = 2
_NS = 16
_NW = _NC * _NS
_BPW = _B // _NW
_PH = 128
_NBUF = 4


def _scalar_at(idx_v, lane, j):
    jm = lax.rem(j, 16)
    grp = j - jm
    v = idx_v[pl.ds(grp, 16)]
    return jnp.max(jnp.where(lane == jm, v, 0))


def _sc_gather_body(uidx_hbm, iidx_hbm, utab_t, itab_t,
                    uout_hbm, iout_hbm,
                    uidx_v, iidx_v, ublk0, ublk1, ublk2, ublk3,
                    iblk0, iblk1, iblk2, iblk3, urows, irows,
                    su0, su1, su2, su3, si0, si1, si2, si3):
    wid = lax.axis_index("s") * _NC + lax.axis_index("c")
    base = wid * _BPW
    pltpu.sync_copy(uidx_hbm.at[pl.ds(base, _BPW)], uidx_v)
    pltpu.sync_copy(iidx_hbm.at[pl.ds(base, _BPW)], iidx_v)
    lane = lax.iota(jnp.int32, 16)
    ublks = [ublk0, ublk1, ublk2, ublk3]
    iblks = [iblk0, iblk1, iblk2, iblk3]
    usems = [su0, su1, su2, su3]
    isems = [si0, si1, si2, si3]

    def fetch(idx_v, tab, blk, sem, hoff, j):
        jc = jnp.minimum(hoff + j, _BPW - 1)
        s = _scalar_at(idx_v, lane, jc)
        c = pl.multiple_of(s - lax.rem(s, 128), 128)

        @pl.when(j < _PH)
        def _():
            pltpu.async_copy(tab.at[:, pl.ds(c, 128)], blk, sem)

        return s

    def extract(s, blk, rows, j):
        l = lax.rem(s, 128)
        lv = jnp.full((16,), 0, jnp.int32) + l
        jv = jnp.full((16,), 0, jnp.int32) + j
        for g in range(4):
            dv = lane + (g * 16)
            vals = plsc.load_gather(blk, [dv, lv])
            plsc.store_scatter(rows, [jv, dv], vals)

    for ph in range(_BPW // _PH):
        hoff = ph * _PH
        scal = []
        for b in range(_NBUF):
            scal.append(fetch(uidx_v, utab_t, ublks[b], usems[b], hoff, b))
            scal.append(fetch(iidx_v, itab_t, iblks[b], isems[b], hoff, b))

        def quad(q, carry):
            nxt = []
            for b in range(_NBUF):
                j = q * _NBUF + b
                pltpu.make_async_copy(utab_t.at[:, pl.ds(0, 128)],
                                      ublks[b], usems[b]).wait()
                extract(carry[2 * b], ublks[b], urows, j)
                pltpu.make_async_copy(itab_t.at[:, pl.ds(0, 128)],
                                      iblks[b], isems[b]).wait()
                extract(carry[2 * b + 1], iblks[b], irows, j)
                nxt.append(fetch(uidx_v, utab_t, ublks[b], usems[b],
                                 hoff, j + _NBUF))
                nxt.append(fetch(iidx_v, itab_t, iblks[b], isems[b],
                                 hoff, j + _NBUF))
            return tuple(nxt)

        lax.fori_loop(0, _PH // _NBUF, quad, tuple(scal))
        pltpu.sync_copy(urows, uout_hbm.at[pl.ds(base + hoff, _PH)])
        pltpu.sync_copy(irows, iout_hbm.at[pl.ds(base + hoff, _PH)])


@functools.cache
def _sc_gather():
    blk = pltpu.VMEM((_D, 128), jnp.float32)
    sem = pltpu.SemaphoreType.DMA
    return pl.kernel(
        _sc_gather_body,
        mesh=plsc.VectorSubcoreMesh(core_axis_name="c", subcore_axis_name="s"),
        compiler_params=pltpu.CompilerParams(needs_layout_passes=False),
        out_type=[
            jax.ShapeDtypeStruct((_B, _D), jnp.float32),
            jax.ShapeDtypeStruct((_B, _D), jnp.float32),
        ],
        scratch_types=[
            pltpu.VMEM((_BPW,), jnp.int32),
            pltpu.VMEM((_BPW,), jnp.int32),
            blk, blk, blk, blk, blk, blk, blk, blk,
            pltpu.VMEM((_PH, _D), jnp.float32),
            pltpu.VMEM((_PH, _D), jnp.float32),
            sem, sem, sem, sem, sem, sem, sem, sem,
        ],
    )


_BLK = 2048


def _mlp_body(u_ref, i_ref, w1u_ref, w1i_ref, b1_ref, w2_ref, b2_ref,
              w3_ref, b3_ref, wo_ref, bo_ref, out_ref):
    f32 = jnp.float32
    h = jnp.dot(u_ref[...], w1u_ref[...], preferred_element_type=f32)
    h += jnp.dot(i_ref[...], w1i_ref[...], preferred_element_type=f32)
    h = jnp.maximum(h + b1_ref[...], 0.0)
    h = jnp.maximum(jnp.dot(h, w2_ref[...], preferred_element_type=f32)
                    + b2_ref[...], 0.0)
    h = jnp.maximum(jnp.dot(h, w3_ref[...], preferred_element_type=f32)
                    + b3_ref[...], 0.0)
    z = jnp.sum(h * wo_ref[...], axis=1, keepdims=True) + bo_ref[...]
    out_ref[...] = 4.0 / (1.0 + jnp.exp(-z)) + 1.0


def _tc_mlp(u_emb, i_emb, w1u, w1i, b1, w2, b2, w3, b3, wo, bo):
    nblk = _B // _BLK
    full = lambda shape: pl.BlockSpec(shape, lambda i: (0, 0))
    return pl.pallas_call(
        _mlp_body,
        grid=(nblk,),
        in_specs=[
            pl.BlockSpec((_BLK, _D), lambda i: (i, 0)),
            pl.BlockSpec((_BLK, _D), lambda i: (i, 0)),
            full((_D, 128)),
            full((_D, 128)),
            full((1, 128)),
            full((128, 64)),
            full((1, 64)),
            full((64, 32)),
            full((1, 32)),
            full((1, 32)),
            full((1, 1)),
        ],
        out_specs=pl.BlockSpec((_BLK, 1), lambda i: (i, 0)),
        out_shape=jax.ShapeDtypeStruct((_B, 1), jnp.float32),
    )(u_emb, i_emb, w1u, w1i, b1, w2, b2, w3, b3, wo, bo)


def kernel(user_indices, item_indices, user_table, item_table,
           W1, b1, W2, b2, W3, b3, Wo, bo):
    u_emb, i_emb = _sc_gather()(user_indices.astype(jnp.int32),
                                item_indices.astype(jnp.int32),
                                user_table.T, item_table.T)
    w1u = W1[:, :_D].T
    w1i = W1[:, _D:].T
    return _tc_mlp(u_emb, i_emb, w1u, w1i,
                   b1.reshape(1, 128), W2.T, b2.reshape(1, 64),
                   W3.T, b3.reshape(1, 32), Wo.reshape(1, 32),
                   bo.reshape(1, 1))


# speedup vs baseline: 2.8734x; 1.0101x over previous
import functools

import jax
import jax.numpy as jnp
from jax import lax
from jax.experimental import pallas as pl
from jax.experimental.pallas import tpu as pltpu
from jax.experimental.pallas import tpu_sc as plsc

_B = 16384
_D = 64
_NC = 2
_NS = 16
_NW = _NC * _NS
_BPW = _B // _NW
_PH = 128
_NBUF = 4


def _scalar_at(idx_v, lane, j):
    jm = lax.rem(j, 16)
    grp = j - jm
    v = idx_v[pl.ds(grp, 16)]
    return jnp.max(jnp.where(lane == jm, v, 0))


def _sc_gather_body(uidx_hbm, iidx_hbm, utab_t, itab_t,
                    uout_hbm, iout_hbm,
                    uidx_v, iidx_v, ublk0, ublk1, ublk2, ublk3,
                    iblk0, iblk1, iblk2, iblk3, urows, irows,
                    su0, su1, su2, su3, si0, si1, si2, si3):
    wid = lax.axis_index("s") * _NC + lax.axis_index("c")
    base = wid * _BPW
    pltpu.sync_copy(uidx_hbm.at[pl.ds(base, _BPW)], uidx_v)
    pltpu.sync_copy(iidx_hbm.at[pl.ds(base, _BPW)], iidx_v)
    lane = lax.iota(jnp.int32, 16)
    ublks = [ublk0, ublk1, ublk2, ublk3]
    iblks = [iblk0, iblk1, iblk2, iblk3]
    usems = [su0, su1, su2, su3]
    isems = [si0, si1, si2, si3]

    def fetch(idx_v, tab, blk, sem, hoff, j):
        jc = jnp.minimum(hoff + j, _BPW - 1)
        s = _scalar_at(idx_v, lane, jc)
        c = pl.multiple_of(s - lax.rem(s, 128), 128)

        @pl.when(j < _PH)
        def _():
            pltpu.async_copy(tab.at[pl.ds(0, 32), pl.ds(c, 128)],
                             blk.at[pl.ds(0, 32)], sem)
            pltpu.async_copy(tab.at[pl.ds(32, 32), pl.ds(c, 128)],
                             blk.at[pl.ds(32, 32)], sem)

        return s

    def extract(s, blk, rows, j):
        l = lax.rem(s, 128)
        lv = jnp.full((16,), 0, jnp.int32) + l
        jv = jnp.full((16,), 0, jnp.int32) + j
        for g in range(4):
            dv = lane + (g * 16)
            vals = plsc.load_gather(blk, [dv, lv])
            plsc.store_scatter(rows, [jv, dv], vals)

    for ph in range(_BPW // _PH):
        hoff = ph * _PH
        scal = []
        for b in range(_NBUF):
            scal.append(fetch(uidx_v, utab_t, ublks[b], usems[b], hoff, b))
            scal.append(fetch(iidx_v, itab_t, iblks[b], isems[b], hoff, b))

        def quad(q, carry):
            nxt = []
            for b in range(_NBUF):
                j = q * _NBUF + b
                pltpu.make_async_copy(utab_t.at[:, pl.ds(0, 128)],
                                      ublks[b], usems[b]).wait()
                extract(carry[2 * b], ublks[b], urows, j)
                pltpu.make_async_copy(itab_t.at[:, pl.ds(0, 128)],
                                      iblks[b], isems[b]).wait()
                extract(carry[2 * b + 1], iblks[b], irows, j)
                nxt.append(fetch(uidx_v, utab_t, ublks[b], usems[b],
                                 hoff, j + _NBUF))
                nxt.append(fetch(iidx_v, itab_t, iblks[b], isems[b],
                                 hoff, j + _NBUF))
            return tuple(nxt)

        lax.fori_loop(0, _PH // _NBUF, quad, tuple(scal))
        pltpu.sync_copy(urows, uout_hbm.at[pl.ds(base + hoff, _PH)])
        pltpu.sync_copy(irows, iout_hbm.at[pl.ds(base + hoff, _PH)])


@functools.cache
def _sc_gather():
    blk = pltpu.VMEM((_D, 128), jnp.float32)
    sem = pltpu.SemaphoreType.DMA
    return pl.kernel(
        _sc_gather_body,
        mesh=plsc.VectorSubcoreMesh(core_axis_name="c", subcore_axis_name="s"),
        compiler_params=pltpu.CompilerParams(needs_layout_passes=False),
        out_type=[
            jax.ShapeDtypeStruct((_B, _D), jnp.float32),
            jax.ShapeDtypeStruct((_B, _D), jnp.float32),
        ],
        scratch_types=[
            pltpu.VMEM((_BPW,), jnp.int32),
            pltpu.VMEM((_BPW,), jnp.int32),
            blk, blk, blk, blk, blk, blk, blk, blk,
            pltpu.VMEM((_PH, _D), jnp.float32),
            pltpu.VMEM((_PH, _D), jnp.float32),
            sem, sem, sem, sem, sem, sem, sem, sem,
        ],
    )


_BLK = 2048


def _mlp_body(u_ref, i_ref, w1u_ref, w1i_ref, b1_ref, w2_ref, b2_ref,
              w3_ref, b3_ref, wo_ref, bo_ref, out_ref):
    f32 = jnp.float32
    h = jnp.dot(u_ref[...], w1u_ref[...], preferred_element_type=f32)
    h += jnp.dot(i_ref[...], w1i_ref[...], preferred_element_type=f32)
    h = jnp.maximum(h + b1_ref[...], 0.0)
    h = jnp.maximum(jnp.dot(h, w2_ref[...], preferred_element_type=f32)
                    + b2_ref[...], 0.0)
    h = jnp.maximum(jnp.dot(h, w3_ref[...], preferred_element_type=f32)
                    + b3_ref[...], 0.0)
    z = jnp.sum(h * wo_ref[...], axis=1, keepdims=True) + bo_ref[...]
    out_ref[...] = 4.0 / (1.0 + jnp.exp(-z)) + 1.0


def _tc_mlp(u_emb, i_emb, w1u, w1i, b1, w2, b2, w3, b3, wo, bo):
    nblk = _B // _BLK
    full = lambda shape: pl.BlockSpec(shape, lambda i: (0, 0))
    return pl.pallas_call(
        _mlp_body,
        grid=(nblk,),
        in_specs=[
            pl.BlockSpec((_BLK, _D), lambda i: (i, 0)),
            pl.BlockSpec((_BLK, _D), lambda i: (i, 0)),
            full((_D, 128)),
            full((_D, 128)),
            full((1, 128)),
            full((128, 64)),
            full((1, 64)),
            full((64, 32)),
            full((1, 32)),
            full((1, 32)),
            full((1, 1)),
        ],
        out_specs=pl.BlockSpec((_BLK, 1), lambda i: (i, 0)),
        out_shape=jax.ShapeDtypeStruct((_B, 1), jnp.float32),
    )(u_emb, i_emb, w1u, w1i, b1, w2, b2, w3, b3, wo, bo)


def kernel(user_indices, item_indices, user_table, item_table,
           W1, b1, W2, b2, W3, b3, Wo, bo):
    u_emb, i_emb = _sc_gather()(user_indices.astype(jnp.int32),
                                item_indices.astype(jnp.int32),
                                user_table.T, item_table.T)
    w1u = W1[:, :_D].T
    w1i = W1[:, _D:].T
    return _tc_mlp(u_emb, i_emb, w1u, w1i,
                   b1.reshape(1, 128), W2.T, b2.reshape(1, 64),
                   W3.T, b3.reshape(1, 32), Wo.reshape(1, 32),
                   bo.reshape(1, 1))
